# Initial kernel scaffold; baseline (speedup 1.0000x reference)
import jax
import jax.numpy as jnp
from jax.experimental import pallas as pl


def kernel(x, edge_index, edge_attr, cW0, cb0, cW1, cb1, cW2, cb2, epW1, epb1, epW2, epb2, fcW1, fcb1, fcW2, fcb2):
    raise NotImplementedError("write your pallas kernel here")


# speedup vs baseline: 5.3470x; 5.3470x over previous
import functools

import jax
import jax.numpy as jnp
from jax import lax
from jax.experimental import pallas as pl
from jax.experimental.pallas import tpu as pltpu
from jax.experimental.pallas import tpu_sc as plsc

_N = 10000
_E = 320000
_HID = 64
_CH = 128
_NC = 2
_NS = 16
_NW = _NC * _NS
_CPT = 80
_EPT = _CPT * _CH
_E_PAD = _NW * _EPT
_ACC = 10240
_RPT = _ACC // _NS

_mesh = plsc.VectorSubcoreMesh(core_axis_name="c", subcore_axis_name="s")
_sc_params = pltpu.CompilerParams(use_tc_tiling_on_sc=False)


def _wid(c, s):
    return c * _NS + s


@functools.partial(
    pl.kernel,
    out_type=jax.ShapeDtypeStruct((_NC, _ACC), jnp.float32),
    mesh=_mesh,
    compiler_params=_sc_params,
    scratch_types=[
        pltpu.VMEM((_CPT, _CH), jnp.int32),
        pltpu.VMEM((_CH,), jnp.float32),
        pltpu.VMEM_SHARED((_ACC,), jnp.float32),
    ],
)
def _sc_degree(dst_hbm, ones_hbm, zeros_hbm, out_hbm, dst_v, ones_v, acc):
    c = lax.axis_index("c")
    s = lax.axis_index("s")
    w = _wid(c, s)
    pltpu.sync_copy(zeros_hbm, acc.at[pl.ds(s * _RPT, _RPT)])
    pltpu.sync_copy(ones_hbm, ones_v)
    pltpu.sync_copy(dst_hbm.at[pl.ds(w * _CPT, _CPT)], dst_v)
    plsc.subcore_barrier()

    def body(j, carry):
        pltpu.sync_copy(ones_v, acc.at[dst_v.at[j]], add=True)
        return carry

    lax.fori_loop(0, _CPT, body, 0)
    plsc.subcore_barrier()
    pltpu.sync_copy(acc.at[pl.ds(s * _RPT, _RPT)], out_hbm.at[c, pl.ds(s * _RPT, _RPT)])


@functools.partial(
    pl.kernel,
    out_type=jax.ShapeDtypeStruct((_NC, _ACC, _HID), jnp.float32),
    mesh=_mesh,
    compiler_params=_sc_params,
    scratch_types=[
        pltpu.VMEM((_CPT, _CH), jnp.int32),
        pltpu.VMEM((_CPT, _CH), jnp.int32),
        pltpu.VMEM((_CH, _HID), jnp.float32),
        pltpu.VMEM_SHARED((_ACC, _HID), jnp.float32),
        pltpu.SemaphoreType.DMA,
    ],
)
def _sc_scatter(y_hbm, src_hbm, dst_hbm, zeros_hbm, out_hbm,
                src_v, dst_v, rows_v, acc, sem):
    c = lax.axis_index("c")
    s = lax.axis_index("s")
    w = _wid(c, s)
    for k in range(_RPT // _CH):
        pltpu.sync_copy(zeros_hbm, acc.at[pl.ds(s * _RPT + k * _CH, _CH)])
    pltpu.sync_copy(src_hbm.at[pl.ds(w * _CPT, _CPT)], src_v)
    pltpu.sync_copy(dst_hbm.at[pl.ds(w * _CPT, _CPT)], dst_v)
    plsc.subcore_barrier()

    def body(j, carry):
        pltpu.async_copy(y_hbm.at[src_v.at[j]], rows_v, sem).wait()
        pltpu.sync_copy(rows_v, acc.at[dst_v.at[j]], add=True)
        return carry

    lax.fori_loop(0, _CPT, body, 0)
    plsc.subcore_barrier()
    pltpu.sync_copy(acc.at[pl.ds(s * _RPT, _RPT)],
                    out_hbm.at[c, pl.ds(s * _RPT, _RPT)])


@functools.partial(
    pl.kernel,
    out_type=[
        jax.ShapeDtypeStruct((_E_PAD, _HID), jnp.float32),
        jax.ShapeDtypeStruct((_E_PAD, _HID), jnp.float32),
    ],
    mesh=_mesh,
    compiler_params=_sc_params,
    scratch_types=[
        pltpu.VMEM((_CPT, _CH), jnp.int32),
        pltpu.VMEM((_CPT, _CH), jnp.int32),
        pltpu.VMEM((_CH, _HID), jnp.float32),
        pltpu.VMEM((_CH, _HID), jnp.float32),
        pltpu.SemaphoreType.DMA,
        pltpu.SemaphoreType.DMA,
    ],
)
def _sc_zgather(zs_hbm, zd_hbm, src_hbm, dst_hbm, os_hbm, od_hbm,
                src_v, dst_v, buf_a, buf_b, sem_a, sem_b):
    c = lax.axis_index("c")
    s = lax.axis_index("s")
    w = _wid(c, s)
    base = w * _EPT
    pltpu.sync_copy(src_hbm.at[pl.ds(w * _CPT, _CPT)], src_v)
    pltpu.sync_copy(dst_hbm.at[pl.ds(w * _CPT, _CPT)], dst_v)

    def body(j, carry):
        ca = pltpu.async_copy(zs_hbm.at[src_v.at[j]], buf_a, sem_a)
        cb = pltpu.async_copy(zd_hbm.at[dst_v.at[j]], buf_b, sem_b)
        ca.wait()
        cb.wait()
        pltpu.sync_copy(buf_a, os_hbm.at[pl.ds(base + j * _CH, _CH)])
        pltpu.sync_copy(buf_b, od_hbm.at[pl.ds(base + j * _CH, _CH)])
        return carry

    lax.fori_loop(0, _CPT, body, 0)


def _dot(a, b):
    return jnp.dot(a, b, preferred_element_type=jnp.float32)


def _tc0_body(degp_ref, x_ref, w_ref, dinv_ref, y0_ref):
    deg = degp_ref[0] + degp_ref[1] + 1.0
    dinv = lax.rsqrt(deg)
    dinv_ref[...] = dinv
    y0_ref[...] = _dot(x_ref[...], w_ref[...]) * dinv


def _tc_layer_body(sp_ref, y_ref, dinv_ref, zs_ref, zd_ref, b_ref,
                   w_ref, bs_ref, bd_ref, yo_ref, zso_ref, zdo_ref):
    dinv = dinv_ref[...]
    h = jnp.maximum(dinv * (sp_ref[0] + sp_ref[1] + y_ref[...]) + b_ref[...], 0.0)
    yo_ref[...] = _dot(h, w_ref[...]) * dinv
    zso_ref[...] = zs_ref[...] + _dot(h, bs_ref[...])
    zdo_ref[...] = zd_ref[...] + _dot(h, bd_ref[...])


def _tc_edge_body(ea_ref, es_ref, ed_ref, w1_ref, c_ref, w2_ref,
                  b1_ref, bf_ref, b2_ref, o_ref):
    t = jnp.maximum(_dot(ea_ref[...], w1_ref[...]) + b1_ref[...], 0.0)
    pre = _dot(t, c_ref[...]) + es_ref[...] + ed_ref[...] + bf_ref[...]
    o_ref[...] = _dot(jnp.maximum(pre, 0.0), w2_ref[...]) + b2_ref[...]


_NB = 125
_NR = 80


def _full(shape):
    return pl.BlockSpec(shape, lambda i: tuple(0 for _ in shape))


def _tc0(degp, x, w):
    return pl.pallas_call(
        _tc0_body,
        grid=(_NB,),
        in_specs=[
            pl.BlockSpec((_NC, _NR, 1), lambda i: (0, i, 0)),
            pl.BlockSpec((_NR, 128), lambda i: (i, 0)),
            _full((128, _HID)),
        ],
        out_specs=[
            pl.BlockSpec((_NR, 1), lambda i: (i, 0)),
            pl.BlockSpec((_NR, _HID), lambda i: (i, 0)),
        ],
        out_shape=[
            jax.ShapeDtypeStruct((_N, 1), jnp.float32),
            jax.ShapeDtypeStruct((_N, _HID), jnp.float32),
        ],
    )(degp, x, w)


def _tc_layer(sp, y, dinv, zs, zd, b, w, bs, bd):
    n64 = pl.BlockSpec((_NR, _HID), lambda i: (i, 0))
    return pl.pallas_call(
        _tc_layer_body,
        grid=(_NB,),
        in_specs=[
            pl.BlockSpec((_NC, _NR, _HID), lambda i: (0, i, 0)),
            n64,
            pl.BlockSpec((_NR, 1), lambda i: (i, 0)),
            n64, n64,
            _full((1, _HID)),
            _full((_HID, _HID)),
            _full((_HID, _HID)),
            _full((_HID, _HID)),
        ],
        out_specs=[n64, n64, n64],
        out_shape=[jax.ShapeDtypeStruct((_N, _HID), jnp.float32)] * 3,
    )(sp, y, dinv, zs, zd, b, w, bs, bd)


_EB = 625
_ER = 512


def _tc_edge(ea, es, ed, w1, cmat, w2, b1, bf, b2):
    e64 = pl.BlockSpec((_ER, _HID), lambda i: (i, 0))
    return pl.pallas_call(
        _tc_edge_body,
        grid=(_EB,),
        in_specs=[
            pl.BlockSpec((_ER, 16), lambda i: (i, 0)),
            e64, e64,
            _full((16, _HID)),
            _full((_HID, _HID)),
            _full((_HID, 16)),
            _full((1, _HID)),
            _full((1, _HID)),
            _full((1, 16)),
        ],
        out_specs=pl.BlockSpec((_ER, 16), lambda i: (i, 0)),
        out_shape=jax.ShapeDtypeStruct((_E, 16), jnp.float32),
    )(ea, es, ed, w1, cmat, w2, b1, bf, b2)


def kernel(x, edge_index, edge_attr, cW0, cb0, cW1, cb1, cW2, cb2,
           epW1, epb1, epW2, epb2, fcW1, fcb1, fcW2, fcb2):
    f32 = jnp.float32
    src = edge_index[0]
    dst = edge_index[1]

    pad = _E_PAD - _E
    src_p = jnp.concatenate([src, jnp.zeros((pad,), jnp.int32)]).reshape(
        _NW * _CPT, _CH)
    dst_p = jnp.concatenate([dst, jnp.full((pad,), _N, jnp.int32)]).reshape(
        _NW * _CPT, _CH)

    ones_ch = jnp.ones((_CH,), f32)
    zeros_row = jnp.zeros((_RPT,), f32)
    zeros_ch64 = jnp.zeros((_CH, _HID), f32)

    degp = _sc_degree(dst_p, ones_ch, zeros_row)
    degp3 = degp.reshape(_NC, _ACC, 1)

    dinv, y0 = _tc0(degp3, x, cW0)

    zeros_n64 = jnp.zeros((_N, _HID), f32)
    sp0 = _sc_scatter(y0, src_p, dst_p, zeros_ch64)
    y1, zs1, zd1 = _tc_layer(sp0, y0, dinv, zeros_n64, zeros_n64,
                             cb0.reshape(1, _HID), cW1,
                             fcW1[0:64], fcW1[64:128])
    sp1 = _sc_scatter(y1, src_p, dst_p, zeros_ch64)
    y2, zs2, zd2 = _tc_layer(sp1, y1, dinv, zs1, zd1,
                             cb1.reshape(1, _HID), cW2,
                             fcW1[128:192], fcW1[192:256])
    sp2 = _sc_scatter(y2, src_p, dst_p, zeros_ch64)
    _, zsrc, zdst = _tc_layer(sp2, y2, dinv, zs2, zd2,
                              cb2.reshape(1, _HID), cW2,
                              fcW1[256:320], fcW1[320:384])

    esrc, edst = _sc_zgather(zsrc, zdst, src_p, dst_p)

    Be = fcW1[384:448]
    cmat = epW2 @ Be
    biasf = (fcb1 + epb2 @ Be).reshape(1, _HID)

    out = _tc_edge(edge_attr, esrc, edst, epW1, cmat, fcW2,
                   epb1.reshape(1, _HID), biasf, fcb2.reshape(1, 16))
    return out


# speedup vs baseline: 5.7668x; 1.0785x over previous
import functools

import jax
import jax.numpy as jnp
from jax import lax
from jax.experimental import pallas as pl
from jax.experimental.pallas import tpu as pltpu
from jax.experimental.pallas import tpu_sc as plsc

_N = 10000
_E = 320000
_HID = 64
_CH = 128
_NC = 2
_NS = 16
_NW = _NC * _NS
_CPT = 80
_EPT = _CPT * _CH
_E_PAD = _NW * _EPT
_ACC = 10240
_RPT = _ACC // _NS

_mesh = plsc.VectorSubcoreMesh(core_axis_name="c", subcore_axis_name="s")
_sc_params = pltpu.CompilerParams(use_tc_tiling_on_sc=False)


def _wid(c, s):
    return c * _NS + s


@functools.partial(
    pl.kernel,
    out_type=jax.ShapeDtypeStruct((_NC, _ACC), jnp.float32),
    mesh=_mesh,
    compiler_params=_sc_params,
    scratch_types=[
        pltpu.VMEM((_CPT, _CH), jnp.int32),
        pltpu.VMEM((_CH,), jnp.float32),
        pltpu.VMEM_SHARED((_ACC,), jnp.float32),
    ],
)
def _sc_degree(dst_hbm, ones_hbm, zeros_hbm, out_hbm, dst_v, ones_v, acc):
    c = lax.axis_index("c")
    s = lax.axis_index("s")
    w = _wid(c, s)
    pltpu.sync_copy(zeros_hbm, acc.at[pl.ds(s * _RPT, _RPT)])
    pltpu.sync_copy(ones_hbm, ones_v)
    pltpu.sync_copy(dst_hbm.at[pl.ds(w * _CPT, _CPT)], dst_v)
    plsc.subcore_barrier()

    def body(j, carry):
        pltpu.sync_copy(ones_v, acc.at[dst_v.at[j]], add=True)
        return carry

    lax.fori_loop(0, _CPT, body, 0)
    plsc.subcore_barrier()
    pltpu.sync_copy(acc.at[pl.ds(s * _RPT, _RPT)], out_hbm.at[c, pl.ds(s * _RPT, _RPT)])


@functools.partial(
    pl.kernel,
    out_type=jax.ShapeDtypeStruct((_NC, _ACC, _HID), jnp.float32),
    mesh=_mesh,
    compiler_params=_sc_params,
    scratch_types=[
        pltpu.VMEM((_CPT, _CH), jnp.int32),
        pltpu.VMEM((_CPT, _CH), jnp.int32),
        pltpu.VMEM((_CH, _HID), jnp.float32),
        pltpu.VMEM((_CH, _HID), jnp.float32),
        pltpu.VMEM_SHARED((_ACC, _HID), jnp.float32),
        pltpu.SemaphoreType.DMA,
        pltpu.SemaphoreType.DMA,
    ],
)
def _sc_scatter(y_hbm, src_hbm, dst_hbm, zeros_hbm, out_hbm,
                src_v, dst_v, rows_a, rows_b, acc, sem_a, sem_b):
    c = lax.axis_index("c")
    s = lax.axis_index("s")
    w = _wid(c, s)
    for k in range(_RPT // _CH):
        pltpu.sync_copy(zeros_hbm, acc.at[pl.ds(s * _RPT + k * _CH, _CH)])
    pltpu.sync_copy(src_hbm.at[pl.ds(w * _CPT, _CPT)], src_v)
    pltpu.sync_copy(dst_hbm.at[pl.ds(w * _CPT, _CPT)], dst_v)
    plsc.subcore_barrier()
    pltpu.async_copy(y_hbm.at[src_v.at[0]], rows_a, sem_a)

    @pl.loop(0, _CPT, step=2)
    def _chunks(j):
        pltpu.async_copy(y_hbm.at[src_v.at[j + 1]], rows_b, sem_b)
        pltpu.make_async_copy(y_hbm.at[src_v.at[j]], rows_a, sem_a).wait()
        pltpu.sync_copy(rows_a, acc.at[dst_v.at[j]], add=True)

        @pl.when(j + 2 < _CPT)
        def _prefetch():
            pltpu.async_copy(y_hbm.at[src_v.at[j + 2]], rows_a, sem_a)

        pltpu.make_async_copy(y_hbm.at[src_v.at[j + 1]], rows_b, sem_b).wait()
        pltpu.sync_copy(rows_b, acc.at[dst_v.at[j + 1]], add=True)

    plsc.subcore_barrier()
    pltpu.sync_copy(acc.at[pl.ds(s * _RPT, _RPT)],
                    out_hbm.at[c, pl.ds(s * _RPT, _RPT)])


@functools.partial(
    pl.kernel,
    out_type=[
        jax.ShapeDtypeStruct((_E_PAD, _HID), jnp.float32),
        jax.ShapeDtypeStruct((_E_PAD, _HID), jnp.float32),
    ],
    mesh=_mesh,
    compiler_params=_sc_params,
    scratch_types=[
        pltpu.VMEM((_CPT, _CH), jnp.int32),
        pltpu.VMEM((_CPT, _CH), jnp.int32),
        pltpu.VMEM((2, _CH, _HID), jnp.float32),
        pltpu.VMEM((2, _CH, _HID), jnp.float32),
        pltpu.SemaphoreType.DMA,
        pltpu.SemaphoreType.DMA,
        pltpu.SemaphoreType.DMA,
        pltpu.SemaphoreType.DMA,
    ],
)
def _sc_zgather(zs_hbm, zd_hbm, src_hbm, dst_hbm, os_hbm, od_hbm,
                src_v, dst_v, buf_a, buf_b, sem_a, sem_b, sem_wa, sem_wb):
    c = lax.axis_index("c")
    s = lax.axis_index("s")
    w = _wid(c, s)
    base = w * _EPT
    pltpu.sync_copy(src_hbm.at[pl.ds(w * _CPT, _CPT)], src_v)
    pltpu.sync_copy(dst_hbm.at[pl.ds(w * _CPT, _CPT)], dst_v)
    pltpu.async_copy(zs_hbm.at[src_v.at[0]], buf_a.at[0], sem_a)
    pltpu.async_copy(zd_hbm.at[dst_v.at[0]], buf_b.at[0], sem_b)

    @pl.loop(0, _CPT, step=2)
    def _chunks(j):
        for p in range(2):
            jj = j + p
            nxt = jj + 1
            @pl.when(jj >= 1)
            def _drain():
                prev = pl.ds(base + (jj - 1) * _CH, _CH)
                pltpu.make_async_copy(buf_a.at[1 - p], os_hbm.at[prev], sem_wa).wait()
                pltpu.make_async_copy(buf_b.at[1 - p], od_hbm.at[prev], sem_wb).wait()

            @pl.when(nxt < _CPT)
            def _prefetch():
                pltpu.async_copy(zs_hbm.at[src_v.at[nxt]], buf_a.at[1 - p], sem_a)
                pltpu.async_copy(zd_hbm.at[dst_v.at[nxt]], buf_b.at[1 - p], sem_b)

            pltpu.make_async_copy(zs_hbm.at[src_v.at[jj]], buf_a.at[p], sem_a).wait()
            pltpu.make_async_copy(zd_hbm.at[dst_v.at[jj]], buf_b.at[p], sem_b).wait()
            dst_slice = pl.ds(base + jj * _CH, _CH)
            pltpu.async_copy(buf_a.at[p], os_hbm.at[dst_slice], sem_wa)
            pltpu.async_copy(buf_b.at[p], od_hbm.at[dst_slice], sem_wb)

    p = (_CPT - 1) % 2
    sl = pl.ds(base + (_CPT - 1) * _CH, _CH)
    pltpu.make_async_copy(buf_a.at[p], os_hbm.at[sl], sem_wa).wait()
    pltpu.make_async_copy(buf_b.at[p], od_hbm.at[sl], sem_wb).wait()


def _dot(a, b):
    return jnp.dot(a, b, preferred_element_type=jnp.float32)


def _tc0_body(degp_ref, x_ref, w_ref, dinv_ref, y0_ref):
    deg = degp_ref[0] + degp_ref[1] + 1.0
    dinv = lax.rsqrt(deg)
    dinv_ref[...] = dinv
    y0_ref[...] = _dot(x_ref[...], w_ref[...]) * dinv


def _tc_layer_body(sp_ref, y_ref, dinv_ref, zs_ref, zd_ref, b_ref,
                   w_ref, bs_ref, bd_ref, yo_ref, zso_ref, zdo_ref):
    dinv = dinv_ref[...]
    h = jnp.maximum(dinv * (sp_ref[0] + sp_ref[1] + y_ref[...]) + b_ref[...], 0.0)
    yo_ref[...] = _dot(h, w_ref[...]) * dinv
    zso_ref[...] = zs_ref[...] + _dot(h, bs_ref[...])
    zdo_ref[...] = zd_ref[...] + _dot(h, bd_ref[...])


def _tc_edge_body(ea_ref, es_ref, ed_ref, w1_ref, c_ref, w2_ref,
                  b1_ref, bf_ref, b2_ref, o_ref):
    t = jnp.maximum(_dot(ea_ref[...], w1_ref[...]) + b1_ref[...], 0.0)
    pre = _dot(t, c_ref[...]) + es_ref[...] + ed_ref[...] + bf_ref[...]
    o_ref[...] = _dot(jnp.maximum(pre, 0.0), w2_ref[...]) + b2_ref[...]


_NB = 125
_NR = 80


def _full(shape):
    return pl.BlockSpec(shape, lambda i: tuple(0 for _ in shape))


def _tc0(degp, x, w):
    return pl.pallas_call(
        _tc0_body,
        grid=(_NB,),
        in_specs=[
            pl.BlockSpec((_NC, _NR, 1), lambda i: (0, i, 0)),
            pl.BlockSpec((_NR, 128), lambda i: (i, 0)),
            _full((128, _HID)),
        ],
        out_specs=[
            pl.BlockSpec((_NR, 1), lambda i: (i, 0)),
            pl.BlockSpec((_NR, _HID), lambda i: (i, 0)),
        ],
        out_shape=[
            jax.ShapeDtypeStruct((_N, 1), jnp.float32),
            jax.ShapeDtypeStruct((_N, _HID), jnp.float32),
        ],
    )(degp, x, w)


def _tc_layer(sp, y, dinv, zs, zd, b, w, bs, bd):
    n64 = pl.BlockSpec((_NR, _HID), lambda i: (i, 0))
    return pl.pallas_call(
        _tc_layer_body,
        grid=(_NB,),
        in_specs=[
            pl.BlockSpec((_NC, _NR, _HID), lambda i: (0, i, 0)),
            n64,
            pl.BlockSpec((_NR, 1), lambda i: (i, 0)),
            n64, n64,
            _full((1, _HID)),
            _full((_HID, _HID)),
            _full((_HID, _HID)),
            _full((_HID, _HID)),
        ],
        out_specs=[n64, n64, n64],
        out_shape=[jax.ShapeDtypeStruct((_N, _HID), jnp.float32)] * 3,
    )(sp, y, dinv, zs, zd, b, w, bs, bd)


_EB = 625
_ER = 512


def _tc_edge(ea, es, ed, w1, cmat, w2, b1, bf, b2):
    e64 = pl.BlockSpec((_ER, _HID), lambda i: (i, 0))
    return pl.pallas_call(
        _tc_edge_body,
        grid=(_EB,),
        in_specs=[
            pl.BlockSpec((_ER, 16), lambda i: (i, 0)),
            e64, e64,
            _full((16, _HID)),
            _full((_HID, _HID)),
            _full((_HID, 16)),
            _full((1, _HID)),
            _full((1, _HID)),
            _full((1, 16)),
        ],
        out_specs=pl.BlockSpec((_ER, 16), lambda i: (i, 0)),
        out_shape=jax.ShapeDtypeStruct((_E, 16), jnp.float32),
    )(ea, es, ed, w1, cmat, w2, b1, bf, b2)


def kernel(x, edge_index, edge_attr, cW0, cb0, cW1, cb1, cW2, cb2,
           epW1, epb1, epW2, epb2, fcW1, fcb1, fcW2, fcb2):
    f32 = jnp.float32
    src = edge_index[0]
    dst = edge_index[1]

    pad = _E_PAD - _E
    src_p = jnp.concatenate([src, jnp.zeros((pad,), jnp.int32)]).reshape(
        _NW * _CPT, _CH)
    dst_p = jnp.concatenate([dst, jnp.full((pad,), _N, jnp.int32)]).reshape(
        _NW * _CPT, _CH)

    ones_ch = jnp.ones((_CH,), f32)
    zeros_row = jnp.zeros((_RPT,), f32)
    zeros_ch64 = jnp.zeros((_CH, _HID), f32)

    degp = _sc_degree(dst_p, ones_ch, zeros_row)
    degp3 = degp.reshape(_NC, _ACC, 1)

    dinv, y0 = _tc0(degp3, x, cW0)

    zeros_n64 = jnp.zeros((_N, _HID), f32)
    sp0 = _sc_scatter(y0, src_p, dst_p, zeros_ch64)
    y1, zs1, zd1 = _tc_layer(sp0, y0, dinv, zeros_n64, zeros_n64,
                             cb0.reshape(1, _HID), cW1,
                             fcW1[0:64], fcW1[64:128])
    sp1 = _sc_scatter(y1, src_p, dst_p, zeros_ch64)
    y2, zs2, zd2 = _tc_layer(sp1, y1, dinv, zs1, zd1,
                             cb1.reshape(1, _HID), cW2,
                             fcW1[128:192], fcW1[192:256])
    sp2 = _sc_scatter(y2, src_p, dst_p, zeros_ch64)
    _, zsrc, zdst = _tc_layer(sp2, y2, dinv, zs2, zd2,
                              cb2.reshape(1, _HID), cW2,
                              fcW1[256:320], fcW1[320:384])

    esrc, edst = _sc_zgather(zsrc, zdst, src_p, dst_p)

    Be = fcW1[384:448]
    cmat = epW2 @ Be
    biasf = (fcb1 + epb2 @ Be).reshape(1, _HID)

    out = _tc_edge(edge_attr, esrc, edst, epW1, cmat, fcW2,
                   epb1.reshape(1, _HID), biasf, fcb2.reshape(1, 16))
    return out


# speedup vs baseline: 6.9199x; 1.2000x over previous
import functools

import jax
import jax.numpy as jnp
from jax import lax
from jax.experimental import pallas as pl
from jax.experimental.pallas import tpu as pltpu
from jax.experimental.pallas import tpu_sc as plsc

_N = 10000
_E = 320000
_HID = 64
_CH = 128
_NC = 2
_NS = 16
_NW = _NC * _NS
_CPT = 80
_EPT = _CPT * _CH
_E_PAD = _NW * _EPT
_ACC = 10240
_RPT = _ACC // _NS
_SS = 4
_NSC = _CPT // _SS
_SCH = _SS * _CH
_SS2 = 2
_NSC2 = _CPT // _SS2
_SCH2 = _SS2 * _CH
_NSC0 = 30
_NSC2_0 = 60
_SPAD = 672
_SPAD2 = 1344

_mesh = plsc.VectorSubcoreMesh(core_axis_name="c", subcore_axis_name="s")
_sc_params = pltpu.CompilerParams(use_tc_tiling_on_sc=False)


def _wid(c, s):
    return c * _NS + s


@functools.partial(
    pl.kernel,
    out_type=jax.ShapeDtypeStruct((_NC, _ACC), jnp.float32),
    mesh=_mesh,
    compiler_params=_sc_params,
    scratch_types=[
        pltpu.VMEM((_EPT,), jnp.int32),
        pltpu.VMEM((_EPT,), jnp.float32),
        pltpu.VMEM_SHARED((_ACC,), jnp.float32),
    ],
)
def _sc_degree(dst_hbm, ones_hbm, zeros_hbm, out_hbm, dst_v, ones_v, acc):
    c = lax.axis_index("c")
    s = lax.axis_index("s")
    w = _wid(c, s)
    pltpu.sync_copy(zeros_hbm, acc.at[pl.ds(s * _RPT, _RPT)])
    pltpu.sync_copy(ones_hbm, ones_v)
    pltpu.sync_copy(dst_hbm.at[pl.ds(w * _EPT, _EPT)], dst_v)
    plsc.subcore_barrier()
    pltpu.sync_copy(ones_v, acc.at[dst_v], add=True)
    plsc.subcore_barrier()
    pltpu.sync_copy(acc.at[pl.ds(s * _RPT, _RPT)], out_hbm.at[c, pl.ds(s * _RPT, _RPT)])


@functools.partial(
    pl.kernel,
    out_type=jax.ShapeDtypeStruct((_NC, _ACC, 2 * _HID), jnp.float32),
    mesh=_mesh,
    compiler_params=_sc_params,
    scratch_types=[
        pltpu.VMEM((_NSC2_0, _SCH2), jnp.int32),
        pltpu.VMEM((_NSC2_0, _SCH2), jnp.int32),
        pltpu.VMEM((_SCH2, _HID), jnp.float32),
        pltpu.VMEM((_SCH2, _HID), jnp.float32),
        pltpu.VMEM_SHARED((_ACC, _HID), jnp.float32),
        pltpu.SemaphoreType.DMA,
        pltpu.SemaphoreType.DMA,
    ],
)
def _sc_scatter(y_hbm, src_hbm, dst_hbm, zeros_hbm, out_hbm,
                src_v, dst_v, rows_a, rows_b, acc, sem_a, sem_b):
    c = lax.axis_index("c")
    s = lax.axis_index("s")
    w = _wid(c, s)
    for k in range(_RPT // _CH):
        pltpu.sync_copy(zeros_hbm, acc.at[pl.ds(s * _RPT + k * _CH, _CH)])
    nsc = 20 + 40 * c
    sbase = c * 320 + s * nsc
    pltpu.sync_copy(src_hbm.at[pl.ds(sbase, _NSC2_0)], src_v)
    pltpu.sync_copy(dst_hbm.at[pl.ds(sbase, _NSC2_0)], dst_v)
    plsc.subcore_barrier()
    pltpu.async_copy(y_hbm.at[src_v.at[0]], rows_a, sem_a)

    @pl.loop(0, nsc, step=2)
    def _chunks(j):
        pltpu.async_copy(y_hbm.at[src_v.at[j + 1]], rows_b, sem_b)
        pltpu.make_async_copy(y_hbm.at[src_v.at[j]], rows_a, sem_a).wait()
        pltpu.sync_copy(rows_a, acc.at[dst_v.at[j]], add=True)

        @pl.when(j + 2 < nsc)
        def _prefetch():
            pltpu.async_copy(y_hbm.at[src_v.at[j + 2]], rows_a, sem_a)

        pltpu.make_async_copy(y_hbm.at[src_v.at[j + 1]], rows_b, sem_b).wait()
        pltpu.sync_copy(rows_b, acc.at[dst_v.at[j + 1]], add=True)

    plsc.subcore_barrier()
    pltpu.sync_copy(acc.at[pl.ds(s * _RPT, _RPT)],
                    out_hbm.at[c, pl.ds(s * _RPT, _RPT), pl.ds(0, _HID)])


@functools.partial(
    pl.kernel,
    out_type=jax.ShapeDtypeStruct((_E_PAD, 2 * _HID), jnp.float32),
    mesh=_mesh,
    compiler_params=_sc_params,
    scratch_types=[
        pltpu.VMEM((_NSC2_0, _SCH2), jnp.int32),
        pltpu.VMEM((_NSC2_0, _SCH2), jnp.int32),
        pltpu.VMEM((2, _SCH2, _HID), jnp.float32),
        pltpu.VMEM((2, _SCH2, _HID), jnp.float32),
        pltpu.SemaphoreType.DMA,
        pltpu.SemaphoreType.DMA,
        pltpu.SemaphoreType.DMA,
    ],
)
def _sc_zgather(zs_hbm, zd_hbm, src_hbm, dst_hbm, os_hbm,
                src_v, dst_v, buf_a, buf_b, sem_a, sem_b, sem_w):
    c = lax.axis_index("c")
    s = lax.axis_index("s")
    w = _wid(c, s)
    nsc2 = 20 + 40 * c
    sbase = c * 320 + s * nsc2
    base = sbase * _SCH2
    pltpu.sync_copy(src_hbm.at[pl.ds(sbase, _NSC2_0)], src_v)
    pltpu.sync_copy(dst_hbm.at[pl.ds(sbase, _NSC2_0)], dst_v)
    pltpu.async_copy(zs_hbm.at[src_v.at[0]], buf_a.at[0], sem_a)
    pltpu.async_copy(zd_hbm.at[dst_v.at[0]], buf_b.at[0], sem_b)

    @pl.loop(0, nsc2, step=2)
    def _chunks(j):
        for p in range(2):
            jj = j + p
            nxt = jj + 1
            @pl.when(jj >= 1)
            def _drain():
                prev = pl.ds(base + (jj - 1) * _SCH2, _SCH2)
                pltpu.make_async_copy(
                    buf_a.at[1 - p], os_hbm.at[prev, pl.ds(0, _HID)], sem_w).wait()
                pltpu.make_async_copy(
                    buf_b.at[1 - p], os_hbm.at[prev, pl.ds(_HID, _HID)], sem_w).wait()

            @pl.when(nxt < nsc2)
            def _prefetch():
                pltpu.async_copy(zs_hbm.at[src_v.at[nxt]], buf_a.at[1 - p], sem_a)
                pltpu.async_copy(zd_hbm.at[dst_v.at[nxt]], buf_b.at[1 - p], sem_b)

            pltpu.make_async_copy(zs_hbm.at[src_v.at[jj]], buf_a.at[p], sem_a).wait()
            pltpu.make_async_copy(zd_hbm.at[dst_v.at[jj]], buf_b.at[p], sem_b).wait()
            dst_rows = pl.ds(base + jj * _SCH2, _SCH2)
            pltpu.async_copy(buf_a.at[p], os_hbm.at[dst_rows, pl.ds(0, _HID)], sem_w)
            pltpu.async_copy(buf_b.at[p], os_hbm.at[dst_rows, pl.ds(_HID, _HID)], sem_w)

    sl = pl.ds(base + (nsc2 - 1) * _SCH2, _SCH2)
    pltpu.make_async_copy(buf_a.at[1], os_hbm.at[sl, pl.ds(0, _HID)], sem_w).wait()
    pltpu.make_async_copy(buf_b.at[1], os_hbm.at[sl, pl.ds(_HID, _HID)], sem_w).wait()


def _dot(a, b):
    return jnp.dot(a, b, preferred_element_type=jnp.float32)


def _tc0_body(degp_ref, x_ref, w_ref, dinv_ref, y0_ref):
    deg = degp_ref[0] + degp_ref[1] + 1.0
    dinv = lax.rsqrt(deg)
    dinv_ref[...] = dinv
    y0_ref[...] = _dot(x_ref[...], w_ref[...]) * dinv


def _tc_layer_body(sp_ref, y_ref, dinv_ref, zs_ref, zd_ref, b_ref,
                   w_ref, bs_ref, bd_ref, yo_ref, zso_ref, zdo_ref):
    dinv = dinv_ref[...]
    sp = sp_ref[0, :, :_HID] + sp_ref[1, :, :_HID]
    h = jnp.maximum(dinv * (sp + y_ref[...]) + b_ref[...], 0.0)
    yo_ref[...] = _dot(h, w_ref[...]) * dinv
    zso_ref[...] = zs_ref[...] + _dot(h, bs_ref[...])
    zdo_ref[...] = zd_ref[...] + _dot(h, bd_ref[...])


def _tc_edge_body(ea_ref, ez_ref, w1_ref, c_ref, w2_ref,
                  b1_ref, bf_ref, b2_ref, o_ref):
    t = jnp.maximum(_dot(ea_ref[...], w1_ref[...]) + b1_ref[...], 0.0)
    u = _dot(t, c_ref[...]) + bf_ref[...]
    ez = ez_ref[...]
    pre = u + ez[:, :_HID] + ez[:, _HID:]
    o_ref[...] = _dot(jnp.maximum(pre, 0.0), w2_ref[...]) + b2_ref[...]


_NB = 5
_NR = 2000


def _full(shape):
    return pl.BlockSpec(shape, lambda i: tuple(0 for _ in shape))


def _tc0(degp, x, w):
    return pl.pallas_call(
        _tc0_body,
        grid=(_NB,),
        in_specs=[
            pl.BlockSpec((_NC, _NR, 1), lambda i: (0, i, 0)),
            pl.BlockSpec((_NR, 128), lambda i: (i, 0)),
            _full((128, _HID)),
        ],
        out_specs=[
            pl.BlockSpec((_NR, 1), lambda i: (i, 0)),
            pl.BlockSpec((_NR, _HID), lambda i: (i, 0)),
        ],
        out_shape=[
            jax.ShapeDtypeStruct((_N, 1), jnp.float32),
            jax.ShapeDtypeStruct((_N, _HID), jnp.float32),
        ],
    )(degp, x, w)


def _tc_layer(sp, y, dinv, zs, zd, b, w, bs, bd):
    n64 = pl.BlockSpec((_NR, _HID), lambda i: (i, 0))
    return pl.pallas_call(
        _tc_layer_body,
        grid=(_NB,),
        in_specs=[
            pl.BlockSpec((_NC, _NR, 2 * _HID), lambda i: (0, i, 0)),
            n64,
            pl.BlockSpec((_NR, 1), lambda i: (i, 0)),
            n64, n64,
            _full((1, _HID)),
            _full((_HID, _HID)),
            _full((_HID, _HID)),
            _full((_HID, _HID)),
        ],
        out_specs=[n64, n64, n64],
        out_shape=[jax.ShapeDtypeStruct((_N, _HID), jnp.float32)] * 3,
    )(sp, y, dinv, zs, zd, b, w, bs, bd)


_EB = 625
_ER = 512


def _tc_edge(ea, ez, w1, cmat, w2, b1, bf, b2):
    return pl.pallas_call(
        _tc_edge_body,
        grid=(_EB,),
        in_specs=[
            pl.BlockSpec((_ER, 16), lambda i: (i, 0)),
            pl.BlockSpec((_ER, 2 * _HID), lambda i: (i, 0)),
            _full((16, _HID)),
            _full((_HID, _HID)),
            _full((_HID, 16)),
            _full((1, _HID)),
            _full((1, _HID)),
            _full((1, 16)),
        ],
        out_specs=pl.BlockSpec((_ER, 16), lambda i: (i, 0)),
        out_shape=jax.ShapeDtypeStruct((_E, 16), jnp.float32),
    )(ea, ez, w1, cmat, w2, b1, bf, b2)


def kernel(x, edge_index, edge_attr, cW0, cb0, cW1, cb1, cW2, cb2,
           epW1, epb1, epW2, epb2, fcW1, fcb1, fcW2, fcb2):
    f32 = jnp.float32
    src = edge_index[0]
    dst = edge_index[1]

    pad = _E_PAD - _E
    src_f = jnp.concatenate([src, jnp.zeros((pad,), jnp.int32)])
    dst_f = jnp.concatenate([dst, jnp.full((pad,), _N, jnp.int32)])
    xpad = jnp.zeros(((_SPAD2 * _SCH2 - _E_PAD),), jnp.int32)
    src_z = jnp.concatenate([src_f, xpad]).reshape(_SPAD2, _SCH2)
    dst_z = jnp.concatenate([dst_f, xpad]).reshape(_SPAD2, _SCH2)

    ones_ch = jnp.ones((_EPT,), f32)
    zeros_row = jnp.zeros((_RPT,), f32)
    zeros_ch64 = jnp.zeros((_CH, _HID), f32)

    degp = _sc_degree(dst_f, ones_ch, zeros_row)
    degp3 = degp[:, :_N].reshape(_NC, _N, 1)

    dinv, y0 = _tc0(degp3, x, cW0)

    zeros_n64 = jnp.zeros((_N, _HID), f32)
    sp0 = _sc_scatter(y0, src_z, dst_z, zeros_ch64)
    y1, zs1, zd1 = _tc_layer(sp0[:, :_N], y0, dinv, zeros_n64, zeros_n64,
                             cb0.reshape(1, _HID), cW1,
                             fcW1[0:64], fcW1[64:128])
    sp1 = _sc_scatter(y1, src_z, dst_z, zeros_ch64)
    y2, zs2, zd2 = _tc_layer(sp1[:, :_N], y1, dinv, zs1, zd1,
                             cb1.reshape(1, _HID), cW2,
                             fcW1[128:192], fcW1[192:256])
    sp2 = _sc_scatter(y2, src_z, dst_z, zeros_ch64)
    _, zsrc, zdst = _tc_layer(sp2[:, :_N], y2, dinv, zs2, zd2,
                              cb2.reshape(1, _HID), cW2,
                              fcW1[256:320], fcW1[320:384])

    Be = fcW1[384:448]
    cmat = epW2 @ Be
    biasf = (fcb1 + epb2 @ Be).reshape(1, _HID)

    ez = _sc_zgather(zsrc, zdst, src_z, dst_z)

    out = _tc_edge(edge_attr, ez, epW1, cmat, fcW2,
                   epb1.reshape(1, _HID), biasf, fcb2.reshape(1, 16))
    return out


# speedup vs baseline: 7.7409x; 1.1186x over previous
import functools

import jax
import jax.numpy as jnp
from jax import lax
from jax.experimental import pallas as pl
from jax.experimental.pallas import tpu as pltpu
from jax.experimental.pallas import tpu_sc as plsc

_N = 10000
_E = 320000
_HID = 64
_CH = 128
_NC = 2
_NS = 16
_NW = _NC * _NS
_CPT = 80
_EPT = _CPT * _CH
_E_PAD = _NW * _EPT
_ACC = 10240
_RPT = _ACC // _NS
_SS = 4
_NSC = _CPT // _SS
_SCH = _SS * _CH
_SS2 = 2
_NSC2 = _CPT // _SS2
_SCH2 = _SS2 * _CH
_NSC0 = 30
_NSC2_0 = 60
_SPAD = 672
_SPAD2 = 1344

_mesh = plsc.VectorSubcoreMesh(core_axis_name="c", subcore_axis_name="s")
_sc_params = pltpu.CompilerParams(use_tc_tiling_on_sc=False)


def _wid(c, s):
    return c * _NS + s


@functools.partial(
    pl.kernel,
    out_type=jax.ShapeDtypeStruct((_NC, _ACC), jnp.float32),
    mesh=_mesh,
    compiler_params=_sc_params,
    scratch_types=[
        pltpu.VMEM((_EPT,), jnp.int32),
        pltpu.VMEM((_EPT,), jnp.float32),
        pltpu.VMEM_SHARED((_ACC,), jnp.float32),
    ],
)
def _sc_degree(dst_hbm, ones_hbm, zeros_hbm, out_hbm, dst_v, ones_v, acc):
    c = lax.axis_index("c")
    s = lax.axis_index("s")
    w = _wid(c, s)
    pltpu.sync_copy(zeros_hbm, acc.at[pl.ds(s * _RPT, _RPT)])
    pltpu.sync_copy(ones_hbm, ones_v)
    pltpu.sync_copy(dst_hbm.at[pl.ds(w * _EPT, _EPT)], dst_v)
    plsc.subcore_barrier()
    pltpu.sync_copy(ones_v, acc.at[dst_v], add=True)
    plsc.subcore_barrier()
    pltpu.sync_copy(acc.at[pl.ds(s * _RPT, _RPT)], out_hbm.at[c, pl.ds(s * _RPT, _RPT)])


@functools.partial(
    pl.kernel,
    out_type=jax.ShapeDtypeStruct((_NC, _ACC, 2 * _HID), jnp.float32),
    mesh=_mesh,
    compiler_params=_sc_params,
    scratch_types=[
        pltpu.VMEM((_NSC, _SCH), jnp.int32),
        pltpu.VMEM((_NSC, _SCH), jnp.int32),
        pltpu.VMEM((_SCH, _HID), jnp.float32),
        pltpu.VMEM((_SCH, _HID), jnp.float32),
        pltpu.VMEM_SHARED((_ACC, _HID), jnp.float32),
        pltpu.SemaphoreType.DMA,
        pltpu.SemaphoreType.DMA,
    ],
)
def _sc_scatter(y_hbm, src_hbm, dst_hbm, zeros_hbm, out_hbm,
                src_v, dst_v, rows_a, rows_b, acc, sem_a, sem_b):
    c = lax.axis_index("c")
    s = lax.axis_index("s")
    w = _wid(c, s)
    for k in range(_RPT // _CH):
        pltpu.sync_copy(zeros_hbm, acc.at[pl.ds(s * _RPT + k * _CH, _CH)])
    pltpu.sync_copy(src_hbm.at[pl.ds(w * _NSC, _NSC)], src_v)
    pltpu.sync_copy(dst_hbm.at[pl.ds(w * _NSC, _NSC)], dst_v)
    plsc.subcore_barrier()
    pltpu.async_copy(y_hbm.at[src_v.at[0]], rows_a, sem_a)

    @pl.loop(0, _NSC, step=2)
    def _chunks(j):
        pltpu.async_copy(y_hbm.at[src_v.at[j + 1]], rows_b, sem_b)
        pltpu.make_async_copy(y_hbm.at[src_v.at[j]], rows_a, sem_a).wait()
        pltpu.sync_copy(rows_a, acc.at[dst_v.at[j]], add=True)

        @pl.when(j + 2 < _NSC)
        def _prefetch():
            pltpu.async_copy(y_hbm.at[src_v.at[j + 2]], rows_a, sem_a)

        pltpu.make_async_copy(y_hbm.at[src_v.at[j + 1]], rows_b, sem_b).wait()
        pltpu.sync_copy(rows_b, acc.at[dst_v.at[j + 1]], add=True)

    plsc.subcore_barrier()
    pltpu.sync_copy(acc.at[pl.ds(s * _RPT, _RPT)],
                    out_hbm.at[c, pl.ds(s * _RPT, _RPT), pl.ds(0, _HID)])


@functools.partial(
    pl.kernel,
    out_type=jax.ShapeDtypeStruct((_E_PAD, 2 * _HID), jnp.float32),
    mesh=_mesh,
    compiler_params=_sc_params,
    scratch_types=[
        pltpu.VMEM((_NSC2_0, _SCH2), jnp.int32),
        pltpu.VMEM((_NSC2_0, _SCH2), jnp.int32),
        pltpu.VMEM((2, _SCH2, _HID), jnp.float32),
        pltpu.VMEM((2, _SCH2, _HID), jnp.float32),
        pltpu.SemaphoreType.DMA,
        pltpu.SemaphoreType.DMA,
        pltpu.SemaphoreType.DMA,
    ],
)
def _sc_zgather(zs_hbm, zd_hbm, src_hbm, dst_hbm, os_hbm,
                src_v, dst_v, buf_a, buf_b, sem_a, sem_b, sem_w):
    c = lax.axis_index("c")
    s = lax.axis_index("s")
    w = _wid(c, s)
    nsc2 = 60 - 40 * c
    sbase = c * 960 + s * nsc2
    base = sbase * _SCH2
    pltpu.sync_copy(src_hbm.at[pl.ds(sbase, _NSC2_0)], src_v)
    pltpu.sync_copy(dst_hbm.at[pl.ds(sbase, _NSC2_0)], dst_v)
    pltpu.async_copy(zs_hbm.at[src_v.at[0]], buf_a.at[0], sem_a)
    pltpu.async_copy(zd_hbm.at[dst_v.at[0]], buf_b.at[0], sem_b)

    @pl.loop(0, nsc2, step=2)
    def _chunks(j):
        for p in range(2):
            jj = j + p
            nxt = jj + 1
            @pl.when(jj >= 1)
            def _drain():
                prev = pl.ds(base + (jj - 1) * _SCH2, _SCH2)
                pltpu.make_async_copy(
                    buf_a.at[1 - p], os_hbm.at[prev, pl.ds(0, _HID)], sem_w).wait()
                pltpu.make_async_copy(
                    buf_b.at[1 - p], os_hbm.at[prev, pl.ds(_HID, _HID)], sem_w).wait()

            @pl.when(nxt < nsc2)
            def _prefetch():
                pltpu.async_copy(zs_hbm.at[src_v.at[nxt]], buf_a.at[1 - p], sem_a)
                pltpu.async_copy(zd_hbm.at[dst_v.at[nxt]], buf_b.at[1 - p], sem_b)

            pltpu.make_async_copy(zs_hbm.at[src_v.at[jj]], buf_a.at[p], sem_a).wait()
            pltpu.make_async_copy(zd_hbm.at[dst_v.at[jj]], buf_b.at[p], sem_b).wait()
            dst_rows = pl.ds(base + jj * _SCH2, _SCH2)
            pltpu.async_copy(buf_a.at[p], os_hbm.at[dst_rows, pl.ds(0, _HID)], sem_w)
            pltpu.async_copy(buf_b.at[p], os_hbm.at[dst_rows, pl.ds(_HID, _HID)], sem_w)

    sl = pl.ds(base + (nsc2 - 1) * _SCH2, _SCH2)
    pltpu.make_async_copy(buf_a.at[1], os_hbm.at[sl, pl.ds(0, _HID)], sem_w).wait()
    pltpu.make_async_copy(buf_b.at[1], os_hbm.at[sl, pl.ds(_HID, _HID)], sem_w).wait()


def _dot(a, b):
    return jnp.dot(a, b, preferred_element_type=jnp.float32)


def _tc0_body(degp_ref, x_ref, w_ref, dinv_ref, y0_ref):
    deg = degp_ref[0] + degp_ref[1] + 1.0
    dinv = lax.rsqrt(deg)
    dinv_ref[...] = dinv
    y0_ref[...] = _dot(x_ref[...], w_ref[...]) * dinv


def _tc_layer_body(sp_ref, y_ref, dinv_ref, zs_ref, zd_ref, b_ref,
                   w_ref, bs_ref, bd_ref, yo_ref, zso_ref, zdo_ref):
    dinv = dinv_ref[...]
    sp = sp_ref[0, :, :_HID] + sp_ref[1, :, :_HID]
    h = jnp.maximum(dinv * (sp + y_ref[...]) + b_ref[...], 0.0)
    yo_ref[...] = _dot(h, w_ref[...]) * dinv
    zso_ref[...] = zs_ref[...] + _dot(h, bs_ref[...])
    zdo_ref[...] = zd_ref[...] + _dot(h, bd_ref[...])


def _tc_edge_body(ea_ref, ez_ref, w1_ref, c_ref, w2_ref,
                  b1_ref, bf_ref, b2_ref, o_ref):
    t = jnp.maximum(_dot(ea_ref[...], w1_ref[...]) + b1_ref[...], 0.0)
    u = _dot(t, c_ref[...]) + bf_ref[...]
    ez = ez_ref[...]
    pre = u + ez[:, :_HID] + ez[:, _HID:]
    o_ref[...] = _dot(jnp.maximum(pre, 0.0), w2_ref[...]) + b2_ref[...]


_NB = 5
_NR = 2000


def _full(shape):
    return pl.BlockSpec(shape, lambda i: tuple(0 for _ in shape))


def _tc0(degp, x, w):
    return pl.pallas_call(
        _tc0_body,
        grid=(_NB,),
        in_specs=[
            pl.BlockSpec((_NC, _NR, 1), lambda i: (0, i, 0)),
            pl.BlockSpec((_NR, 128), lambda i: (i, 0)),
            _full((128, _HID)),
        ],
        out_specs=[
            pl.BlockSpec((_NR, 1), lambda i: (i, 0)),
            pl.BlockSpec((_NR, _HID), lambda i: (i, 0)),
        ],
        out_shape=[
            jax.ShapeDtypeStruct((_N, 1), jnp.float32),
            jax.ShapeDtypeStruct((_N, _HID), jnp.float32),
        ],
    )(degp, x, w)


def _tc_layer(sp, y, dinv, zs, zd, b, w, bs, bd):
    n64 = pl.BlockSpec((_NR, _HID), lambda i: (i, 0))
    return pl.pallas_call(
        _tc_layer_body,
        grid=(_NB,),
        in_specs=[
            pl.BlockSpec((_NC, _NR, 2 * _HID), lambda i: (0, i, 0)),
            n64,
            pl.BlockSpec((_NR, 1), lambda i: (i, 0)),
            n64, n64,
            _full((1, _HID)),
            _full((_HID, _HID)),
            _full((_HID, _HID)),
            _full((_HID, _HID)),
        ],
        out_specs=[n64, n64, n64],
        out_shape=[jax.ShapeDtypeStruct((_N, _HID), jnp.float32)] * 3,
    )(sp, y, dinv, zs, zd, b, w, bs, bd)


_EB = 625
_ER = 512


def _tc_edge(ea, ez, w1, cmat, w2, b1, bf, b2):
    return pl.pallas_call(
        _tc_edge_body,
        grid=(_EB,),
        in_specs=[
            pl.BlockSpec((_ER, 16), lambda i: (i, 0)),
            pl.BlockSpec((_ER, 2 * _HID), lambda i: (i, 0)),
            _full((16, _HID)),
            _full((_HID, _HID)),
            _full((_HID, 16)),
            _full((1, _HID)),
            _full((1, _HID)),
            _full((1, 16)),
        ],
        out_specs=pl.BlockSpec((_ER, 16), lambda i: (i, 0)),
        out_shape=jax.ShapeDtypeStruct((_E, 16), jnp.float32),
    )(ea, ez, w1, cmat, w2, b1, bf, b2)


def kernel(x, edge_index, edge_attr, cW0, cb0, cW1, cb1, cW2, cb2,
           epW1, epb1, epW2, epb2, fcW1, fcb1, fcW2, fcb2):
    f32 = jnp.float32
    src = edge_index[0]
    dst = edge_index[1]

    pad = _E_PAD - _E
    src_f = jnp.concatenate([src, jnp.zeros((pad,), jnp.int32)])
    dst_f = jnp.concatenate([dst, jnp.full((pad,), _N, jnp.int32)])
    xpad = jnp.zeros(((_SPAD2 * _SCH2 - _E_PAD),), jnp.int32)
    src_z = jnp.concatenate([src_f, xpad]).reshape(_SPAD2, _SCH2)
    dst_z = jnp.concatenate([dst_f, xpad]).reshape(_SPAD2, _SCH2)
    src_s = src_f.reshape(_NW * _NSC, _SCH)
    dst_s = dst_f.reshape(_NW * _NSC, _SCH)

    ones_ch = jnp.ones((_EPT,), f32)
    zeros_row = jnp.zeros((_RPT,), f32)
    zeros_ch64 = jnp.zeros((_CH, _HID), f32)

    degp = _sc_degree(dst_f, ones_ch, zeros_row)
    degp3 = degp[:, :_N].reshape(_NC, _N, 1)

    dinv, y0 = _tc0(degp3, x, cW0)

    zeros_n64 = jnp.zeros((_N, _HID), f32)
    sp0 = _sc_scatter(y0, src_s, dst_s, zeros_ch64)
    y1, zs1, zd1 = _tc_layer(sp0[:, :_N], y0, dinv, zeros_n64, zeros_n64,
                             cb0.reshape(1, _HID), cW1,
                             fcW1[0:64], fcW1[64:128])
    sp1 = _sc_scatter(y1, src_s, dst_s, zeros_ch64)
    y2, zs2, zd2 = _tc_layer(sp1[:, :_N], y1, dinv, zs1, zd1,
                             cb1.reshape(1, _HID), cW2,
                             fcW1[128:192], fcW1[192:256])
    sp2 = _sc_scatter(y2, src_s, dst_s, zeros_ch64)
    _, zsrc, zdst = _tc_layer(sp2[:, :_N], y2, dinv, zs2, zd2,
                              cb2.reshape(1, _HID), cW2,
                              fcW1[256:320], fcW1[320:384])

    Be = fcW1[384:448]
    cmat = epW2 @ Be
    biasf = (fcb1 + epb2 @ Be).reshape(1, _HID)

    ez = _sc_zgather(zsrc, zdst, src_z, dst_z)

    out = _tc_edge(edge_attr, ez, epW1, cmat, fcW2,
                   epb1.reshape(1, _HID), biasf, fcb2.reshape(1, 16))
    return out
